# baseline (speedup 1.0000x reference)
import functools

import jax
import jax.numpy as jnp
from jax import lax
from jax.experimental import pallas as pl
from jax.experimental.pallas import tpu as pltpu
from jax.experimental.pallas import tpu_sc as plsc

_NT = 2048


def _cdist_body(src_ref, dst_ref, dist_f_ref, idx_f_ref, dist_b_ref,
                idx_b_ref):
    nb, _, m = src_ref.shape
    n = dst_ref.shape[2]
    f32, i32 = jnp.float32, jnp.int32

    def step(bb, carry):
        sl = pl.ds(bb, 1)
        xs = src_ref[sl][0]
        ys = dst_ref[sl][0]

        x0 = jnp.reshape(xs[0:1, :], (m, 1))
        x1 = jnp.reshape(xs[1:2, :], (m, 1))
        x2 = jnp.reshape(xs[2:3, :], (m, 1))
        d0 = x0 - ys[0:1, :]
        d1 = x1 - ys[1:2, :]
        d2_ = x2 - ys[2:3, :]
        s = d0 * d0 + d1 * d1 + d2_ * d2_

        bits = lax.bitcast_convert_type(s, i32)
        km = bits & i32(-2048)

        iota_c = lax.broadcasted_iota(i32, (m, n), 1)
        keyr = lax.bitcast_convert_type(km | iota_c, f32)
        rk = lax.bitcast_convert_type(
            jnp.min(keyr, axis=1, keepdims=True), i32)
        rval = lax.bitcast_convert_type(rk & i32(-2048), f32)
        dist_f_ref[sl, :] = jnp.reshape(
            jnp.sqrt(jnp.maximum(rval, 0.0)), (1, m))
        idx_f_ref[sl, :] = jnp.reshape((rk & 0x7FF) + bb * n, (1, m))

        iota_r = lax.broadcasted_iota(i32, (m, n), 0)
        keyc = lax.bitcast_convert_type(km | iota_r, f32)
        ckey = lax.bitcast_convert_type(
            jnp.min(keyc, axis=0, keepdims=True), i32)
        cval = lax.bitcast_convert_type(ckey & i32(-2048), f32)
        dist_b_ref[sl, :] = jnp.sqrt(jnp.maximum(cval, 0.0))
        idx_b_ref[sl, :] = (ckey & 0x7FF) + bb * m
        return carry

    lax.fori_loop(0, nb, step, 0)


def _cdist_call(src, dst, *, interpret=False):
    b, k, m = src.shape
    n = dst.shape[2]
    assert n <= 2048 and m <= 2048
    return pl.pallas_call(
        _cdist_body,
        in_specs=[
            pl.BlockSpec((b, k, m), lambda: (0, 0, 0)),
            pl.BlockSpec((b, k, n), lambda: (0, 0, 0)),
        ],
        out_specs=[
            pl.BlockSpec((b, m), lambda: (0, 0)),
            pl.BlockSpec((b, m), lambda: (0, 0)),
            pl.BlockSpec((b, n), lambda: (0, 0)),
            pl.BlockSpec((b, n), lambda: (0, 0)),
        ],
        out_shape=[
            jax.ShapeDtypeStruct((b, m), jnp.float32),
            jax.ShapeDtypeStruct((b, m), jnp.int32),
            jax.ShapeDtypeStruct((b, n), jnp.float32),
            jax.ShapeDtypeStruct((b, n), jnp.int32),
        ],
        interpret=interpret,
    )(src, dst)


_IW = 128


def _make_sc_gather(b, m, n):
    info = plsc.get_sparse_core_info()
    nc, ns, lanes = info.num_cores, info.num_subcores, info.num_lanes
    nw = nc * ns
    fw = (b * m) // nw
    bw = (b * n) // nw
    assert (b * m) % (nw * _IW) == 0 and (b * n) % (nw * _IW) == 0
    kf = fw // _IW
    kb = bw // _IW
    f_scale = 0.5 / (b * m)
    b_scale = 0.5 / (b * n)
    mesh = plsc.VectorSubcoreMesh(core_axis_name="c", subcore_axis_name="s")

    @functools.partial(
        pl.kernel, mesh=mesh,
        out_type=jax.ShapeDtypeStruct((nw, lanes), jnp.float32),
        scratch_types=[
            pltpu.VMEM((fw,), jnp.int32),
            pltpu.VMEM((fw,), jnp.float32),
            pltpu.VMEM((fw,), jnp.float32),
            pltpu.VMEM((fw,), jnp.float32),
            pltpu.VMEM((bw,), jnp.int32),
            pltpu.VMEM((bw,), jnp.float32),
            pltpu.VMEM((bw,), jnp.float32),
            pltpu.VMEM((bw,), jnp.float32),
            pltpu.VMEM((lanes,), jnp.float32),
            pltpu.SemaphoreType.DMA,
            pltpu.SemaphoreType.DMA,
            pltpu.SemaphoreType.DMA,
        ],
    )
    def sc_fn(df_hbm, if_hbm, ss_hbm, db_hbm, ib_hbm, sd_hbm, out_hbm,
              fidx_v, fgth_v, fdat_v, fsig_v,
              bidx_v, bgth_v, bdat_v, bsig_v, acc_v,
              sem_i, sem_d, sem_g):
        wid = lax.axis_index("s") * nc + lax.axis_index("c")
        fbase = wid * fw
        bbase = wid * bw

        ci_f = pltpu.async_copy(if_hbm.at[pl.ds(fbase, fw)], fidx_v, sem_i)
        ci_b = pltpu.async_copy(ib_hbm.at[pl.ds(bbase, bw)], bidx_v, sem_i)
        cd = [
            pltpu.async_copy(df_hbm.at[pl.ds(fbase, fw)], fdat_v, sem_d),
            pltpu.async_copy(ss_hbm.at[pl.ds(fbase, fw)], fsig_v, sem_d),
            pltpu.async_copy(db_hbm.at[pl.ds(bbase, bw)], bdat_v, sem_d),
            pltpu.async_copy(sd_hbm.at[pl.ds(bbase, bw)], bsig_v, sem_d),
        ]
        ci_f.wait()
        ci_b.wait()
        cg = []
        for k in range(kf):
            sl = pl.ds(k * _IW, _IW)
            cg.append(pltpu.async_copy(sd_hbm.at[fidx_v.at[sl]],
                                       fgth_v.at[sl], sem_g))
        for k in range(kb):
            sl = pl.ds(k * _IW, _IW)
            cg.append(pltpu.async_copy(ss_hbm.at[bidx_v.at[sl]],
                                       bgth_v.at[sl], sem_g))
        for c in cd:
            c.wait()
        for c in cg:
            c.wait()

        acc_f = jnp.zeros((lanes,), jnp.float32)
        for c in range(fw // lanes):
            sl = pl.ds(c * lanes, lanes)
            acc_f = acc_f + fdat_v[sl] * (fsig_v[sl] + fgth_v[sl])

        acc_b = jnp.zeros((lanes,), jnp.float32)
        for c in range(bw // lanes):
            sl = pl.ds(c * lanes, lanes)
            acc_b = acc_b + bdat_v[sl] * (bsig_v[sl] + bgth_v[sl])

        acc_v[...] = acc_f * f_scale + acc_b * b_scale
        pltpu.sync_copy(acc_v, out_hbm.at[wid])

    return sc_fn


def _finalize_body(p_ref, o_ref):
    o_ref[...] = jnp.full((1, 1), jnp.sum(p_ref[...]), jnp.float32)


def _finalize(parts, *, interpret=False):
    return pl.pallas_call(
        _finalize_body,
        out_shape=jax.ShapeDtypeStruct((1, 1), jnp.float32),
        interpret=interpret,
    )(parts)


def kernel(pc_src, pc_dst, sigma_src, sigma_dst):
    b, _, m = pc_src.shape
    n = pc_dst.shape[2]
    dist_f, idx_f, dist_b, idx_b = _cdist_call(pc_src, pc_dst)
    sc_fn = _make_sc_gather(b, m, n)
    parts = sc_fn(dist_f.reshape(-1), idx_f.reshape(-1),
                  sigma_src.reshape(-1), dist_b.reshape(-1),
                  idx_b.reshape(-1), sigma_dst.reshape(-1))
    return _finalize(parts)[0, 0]
